# baseline (speedup 1.0000x reference)
import functools

import jax
import jax.numpy as jnp
from jax import lax
from jax.experimental import pallas as pl
from jax.experimental.pallas import tpu as pltpu
from jax.experimental.pallas import tpu_sc as plsc

NUM_CORES = 2
NUM_SUBCORES = 16
NUM_WORKERS = NUM_CORES * NUM_SUBCORES
BATCH = 16384
EMBED_DIM = 64
BPW = BATCH // NUM_WORKERS
CHUNK = 128
NCHUNK = BPW // CHUNK
IDX_ROWS_PER_W = BPW // CHUNK


def _sc_body(head_h, rel_h, tail_h, ent_h, relemb_h, out_h,
             hidx, ridx, tidx, hrows, rrows, trows, outv, sem):
    wid = lax.axis_index("s") * NUM_CORES + lax.axis_index("c")
    rbase = wid * IDX_ROWS_PER_W

    pltpu.sync_copy(head_h.at[pl.ds(rbase, IDX_ROWS_PER_W)], hidx)
    pltpu.sync_copy(rel_h.at[pl.ds(rbase, IDX_ROWS_PER_W)], ridx)
    pltpu.sync_copy(tail_h.at[pl.ds(rbase, IDX_ROWS_PER_W)], tidx)

    cps = []
    for j in range(NCHUNK):
        cps.append(pltpu.async_copy(
            ent_h.at[hidx.at[j]], hrows.at[pl.ds(j * CHUNK, CHUNK)], sem))
        cps.append(pltpu.async_copy(
            relemb_h.at[ridx.at[j]], rrows.at[pl.ds(j * CHUNK, CHUNK)], sem))
        cps.append(pltpu.async_copy(
            ent_h.at[tidx.at[j]], trows.at[pl.ds(j * CHUNK, CHUNK)], sem))
    for cp in cps:
        cp.wait()

    lanes = lax.iota(jnp.int32, 16)

    dnums = lax.GatherDimensionNumbers(
        offset_dims=(), collapsed_slice_dims=(0,), start_index_map=(0,))

    def lane_sum(v):
        for s in (8, 4, 2, 1):
            perm = lax.gather(
                v, (lanes ^ s)[:, None], dimension_numbers=dnums,
                slice_sizes=(1,),
                mode=lax.GatherScatterMode.PROMISE_IN_BOUNDS)
            v = v + perm
        return v

    def group(g, carry):
        base = g * 16
        scores = jnp.zeros((16,), jnp.float32)
        for j in range(16):
            b = base + j
            acc = (hrows[b, pl.ds(0, 16)] * rrows[b, pl.ds(0, 16)]
                   * trows[b, pl.ds(0, 16)])
            for c in range(1, EMBED_DIM // 16):
                acc = acc + (hrows[b, pl.ds(c * 16, 16)]
                             * rrows[b, pl.ds(c * 16, 16)]
                             * trows[b, pl.ds(c * 16, 16)])
            scores = jnp.where(lanes == j, lane_sum(acc), scores)
        outv[pl.ds(base, 16)] = scores
        return carry

    lax.fori_loop(0, BPW // 16, group, 0)

    pltpu.sync_copy(outv, out_h.at[pl.ds(wid * BPW, BPW)])


@jax.jit
def kernel(head, relation, tail, entity_embeddings, relation_embeddings):
    h = head.astype(jnp.int32).reshape(BATCH // CHUNK, CHUNK)
    r = relation.astype(jnp.int32).reshape(BATCH // CHUNK, CHUNK)
    t = tail.astype(jnp.int32).reshape(BATCH // CHUNK, CHUNK)

    mesh = plsc.VectorSubcoreMesh(core_axis_name="c", subcore_axis_name="s")
    run = functools.partial(
        pl.kernel,
        mesh=mesh,
        compiler_params=pltpu.CompilerParams(use_tc_tiling_on_sc=False),
        out_type=jax.ShapeDtypeStruct((BATCH,), jnp.float32),
        scratch_types=[
            pltpu.VMEM((NCHUNK, CHUNK), jnp.int32),
            pltpu.VMEM((NCHUNK, CHUNK), jnp.int32),
            pltpu.VMEM((NCHUNK, CHUNK), jnp.int32),
            pltpu.VMEM((BPW, EMBED_DIM), jnp.float32),
            pltpu.VMEM((BPW, EMBED_DIM), jnp.float32),
            pltpu.VMEM((BPW, EMBED_DIM), jnp.float32),
            pltpu.VMEM((BPW,), jnp.float32),
            pltpu.SemaphoreType.DMA,
        ],
    )(_sc_body)
    return run(h, r, t, entity_embeddings, relation_embeddings)
